# baseline (speedup 1.0000x reference)
import functools

import jax
import jax.numpy as jnp
from jax import lax
from jax.experimental import pallas as pl
from jax.experimental.pallas import tpu as pltpu
from jax.experimental.pallas import tpu_sc as plsc

N_NODES = 10000
E_EDGES = 160000
IN_CH = 32
OUT_CH = 32
D_EDGE = 16
HID = 128

_NC = 2
_NS = 16
_NW = _NC * _NS
_CHUNK = 128
_E_PAD = 163840
_N_CHUNKS = 5
_CE = _E_PAD // _N_CHUNKS
_N_ACC = 10240
_N_HALF = _N_ACC // _NC
_N_DUMP = 8
_ROWS_PER_SUB = _N_HALF // _NS

_T_E = 256
_T_N = 1024


def _sc_gather(x128, col2d, n_edges):
    nch = n_edges // (_NW * _CHUNK)
    ew = nch * _CHUNK
    mesh = plsc.VectorSubcoreMesh(core_axis_name="c", subcore_axis_name="s")

    @functools.partial(
        pl.kernel,
        mesh=mesh,
        out_type=jax.ShapeDtypeStruct((n_edges, 128), jnp.float32),
        scratch_types=[
            pltpu.VMEM((nch, _CHUNK), jnp.int32),
            pltpu.VMEM((2, _CHUNK, 128), jnp.float32),
            pltpu.SemaphoreType.DMA,
            pltpu.SemaphoreType.DMA,
            pltpu.SemaphoreType.DMA,
        ],
    )
    def gather_kernel(x_hbm, col_hbm, out_hbm, idx_a, rows2, sem_g, sem_w0, sem_w1):
        wid = lax.axis_index("s") * _NC + lax.axis_index("c")
        base = wid * ew
        pltpu.sync_copy(col_hbm.at[pl.ds(wid * nch, nch)], idx_a)
        sems = (sem_w0, sem_w1)

        def step(j, b):
            @pl.when(j >= 2)
            def _():
                pltpu.make_async_copy(
                    rows2.at[b], out_hbm.at[pl.ds(base, _CHUNK)], sems[b]).wait()
            pltpu.async_copy(x_hbm.at[idx_a.at[j]], rows2.at[b], sem_g).wait()
            pltpu.async_copy(
                rows2.at[b], out_hbm.at[pl.ds(base + j * _CHUNK, _CHUNK)], sems[b])

        def body(jj, carry):
            step(2 * jj, 0)
            step(2 * jj + 1, 1)
            return carry

        lax.fori_loop(0, nch // 2, body, 0)
        pltpu.make_async_copy(rows2.at[0], out_hbm.at[pl.ds(base, _CHUNK)], sem_w0).wait()
        pltpu.make_async_copy(rows2.at[1], out_hbm.at[pl.ds(base, _CHUNK)], sem_w1).wait()

    return gather_kernel(x128, col2d)


def _sc_scatter(msgs, row2d, zeros):
    nch = _CE // (_NS * _CHUNK)
    ew = nch * _CHUNK
    rows_per_chunk = _CE // _CHUNK
    nm = _N_CHUNKS
    mesh = plsc.VectorSubcoreMesh(core_axis_name="c", subcore_axis_name="s")

    @functools.partial(
        pl.kernel,
        mesh=mesh,
        out_type=jax.ShapeDtypeStruct((_NC, _N_HALF, 128), jnp.float32),
        scratch_types=[
            pltpu.VMEM((nm * nch, _CHUNK), jnp.int32),
            pltpu.VMEM((4, _CHUNK, 128), jnp.float32),
            pltpu.VMEM_SHARED((_N_HALF + _N_DUMP, 128), jnp.float32),
        ] + [pltpu.SemaphoreType.DMA] * 8,
    )
    def scatter_kernel(m0, m1, m2, m3, m4, row_hbm, z_hbm, out_hbm,
                       idx_a, msg4, acc_sh,
                       sl0, sl1, sl2, sl3, sa0, sa1, sa2, sa3):
        msg_refs = (m0, m1, m2, m3, m4)
        sem_l = (sl0, sl1, sl2, sl3)
        sem_a = (sa0, sa1, sa2, sa3)
        cid = lax.axis_index("c")
        sid = lax.axis_index("s")
        r0 = sid * _ROWS_PER_SUB
        ebase = sid * ew
        pltpu.sync_copy(z_hbm.at[pl.ds(r0, _ROWS_PER_SUB)],
                        acc_sh.at[pl.ds(r0, _ROWS_PER_SUB)])
        rbase = cid * (nm * rows_per_chunk)
        for m in range(nm):
            pltpu.sync_copy(
                row_hbm.at[pl.ds(rbase + m * rows_per_chunk + sid * nch, nch)],
                idx_a.at[pl.ds(m * nch, nch)])
        plsc.subcore_barrier()

        def start_load(mref, t, b):
            pltpu.async_copy(mref.at[pl.ds(ebase + t * _CHUNK, _CHUNK)],
                             msg4.at[b], sem_l[b])

        def wait_load(b):
            pltpu.make_async_copy(m0.at[pl.ds(0, _CHUNK)], msg4.at[b],
                                  sem_l[b]).wait()

        def start_add(m, t2, b):
            pltpu.async_copy(msg4.at[b], acc_sh.at[idx_a.at[m * nch + t2]],
                             sem_a[b], add=True)

        def wait_add(b):
            pltpu.make_async_copy(msg4.at[b], acc_sh.at[idx_a.at[0]],
                                  sem_a[b]).wait()

        for m in range(nm):
            def body(jj, carry, m=m):
                for db in range(4):
                    t = 4 * jj + db
                    if m > 0:
                        wait_add(db)
                    else:
                        @pl.when(jj >= 1)
                        def _(db=db):
                            wait_add(db)
                    start_load(msg_refs[m], t, db)
                    if db < 2:
                        bb = db + 2
                        if m > 0:
                            @pl.when(jj == 0)
                            def _(db=db, bb=bb, m=m):
                                wait_load(bb)
                                start_add(m - 1, nch - 2 + db, bb)
                        @pl.when(jj >= 1)
                        def _(jj_=None, db=db, bb=bb, m=m):
                            wait_load(bb)
                            start_add(m, 4 * jj + db - 2, bb)
                    else:
                        bb = db - 2
                        wait_load(bb)
                        start_add(m, 4 * jj + db - 2, bb)
                return carry

            lax.fori_loop(0, nch // 4, body, 0)

        wait_load(2)
        start_add(nm - 1, nch - 2, 2)
        wait_load(3)
        start_add(nm - 1, nch - 1, 3)
        for b in range(4):
            wait_add(b)

        plsc.subcore_barrier()
        pltpu.sync_copy(acc_sh.at[pl.ds(r0, _ROWS_PER_SUB)],
                        out_hbm.at[cid, pl.ds(r0, _ROWS_PER_SUB)])

    return scatter_kernel(msgs[0], msgs[1], msgs[2], msgs[3], msgs[4], row2d, zeros)


def _msg_body(ps_ref, xj_ref, w1_ref, b1_ref, w2_ref, b2_ref, exp_ref, out_ref):
    ps = ps_ref[...]
    h = jnp.maximum(
        jnp.dot(ps, w1_ref[...], preferred_element_type=jnp.float32) + b1_ref[...],
        0.0)
    xj = xj_ref[...][:, :IN_CH]
    xr = jnp.dot(xj.astype(jnp.bfloat16), exp_ref[...],
                 preferred_element_type=jnp.float32).astype(jnp.bfloat16)
    hr = jnp.broadcast_to(h.astype(jnp.bfloat16)[:, None, :],
                          (_T_E, IN_CH, HID)).reshape(_T_E, IN_CH * HID)
    msg = jnp.dot(xr * hr, w2_ref[...], preferred_element_type=jnp.float32)
    msg = msg + jnp.dot(xj, b2_ref[...], preferred_element_type=jnp.float32)
    out_ref[...] = jnp.concatenate(
        [msg, jnp.zeros((_T_E, 128 - OUT_CH), jnp.float32)], axis=1)


def _tc_messages(pseudo_c, x_j, W1, b1, W2m, b2m, exp, n_edges, interpret=False):
    grid = n_edges // _T_E
    return pl.pallas_call(
        _msg_body,
        grid=(grid,),
        in_specs=[
            pl.BlockSpec((_T_E, D_EDGE), lambda i: (i, 0)),
            pl.BlockSpec((_T_E, 128), lambda i: (i, 0)),
            pl.BlockSpec((D_EDGE, HID), lambda i: (0, 0)),
            pl.BlockSpec((1, HID), lambda i: (0, 0)),
            pl.BlockSpec((IN_CH * HID, OUT_CH), lambda i: (0, 0)),
            pl.BlockSpec((IN_CH, OUT_CH), lambda i: (0, 0)),
            pl.BlockSpec((IN_CH, IN_CH * HID), lambda i: (0, 0)),
        ],
        out_specs=pl.BlockSpec((_T_E, 128), lambda i: (i, 0)),
        out_shape=jax.ShapeDtypeStruct((n_edges, 128), jnp.float32),
        interpret=interpret,
    )(pseudo_c, x_j, W1, b1.reshape(1, HID), W2m, b2m, exp)


def _combine_body(p_ref, x_ref, root_ref, bias_ref, out_ref):
    acc = p_ref[...][:, :OUT_CH]
    acc = acc + jnp.dot(x_ref[...], root_ref[...], preferred_element_type=jnp.float32)
    out_ref[...] = acc + bias_ref[...]


def _tc_combine(p, x_pad, root, bias, interpret=False):
    grid = _N_ACC // _T_N
    return pl.pallas_call(
        _combine_body,
        grid=(grid,),
        in_specs=[
            pl.BlockSpec((_T_N, 128), lambda i: (i, 0)),
            pl.BlockSpec((_T_N, IN_CH), lambda i: (i, 0)),
            pl.BlockSpec((IN_CH, OUT_CH), lambda i: (0, 0)),
            pl.BlockSpec((1, OUT_CH), lambda i: (0, 0)),
        ],
        out_specs=pl.BlockSpec((_T_N, OUT_CH), lambda i: (i, 0)),
        out_shape=jax.ShapeDtypeStruct((_N_ACC, OUT_CH), jnp.float32),
        interpret=interpret,
    )(p, x_pad, root, bias.reshape(1, OUT_CH))


def kernel(x, edge_index, pseudo, W1, b1, W2, b2, root, bias):
    row = edge_index[0]
    col = edge_index[1]
    pad_e = _E_PAD - E_EDGES
    col_p = jnp.concatenate([col, jnp.zeros((pad_e,), jnp.int32)])
    row_p = jnp.concatenate([row, jnp.full((pad_e,), N_NODES, jnp.int32)])
    pseudo_p = jnp.concatenate(
        [pseudo, jnp.zeros((pad_e, D_EDGE), jnp.float32)])
    W2m = (W2.reshape(HID, IN_CH, OUT_CH).transpose(1, 0, 2)
           .reshape(IN_CH * HID, OUT_CH).astype(jnp.bfloat16))
    b2m = b2.reshape(IN_CH, OUT_CH)
    exp = (jnp.arange(IN_CH * HID, dtype=jnp.int32)[None, :] // HID
           == jnp.arange(IN_CH, dtype=jnp.int32)[:, None]).astype(jnp.bfloat16)
    zeros = jnp.zeros((_N_HALF, 128), jnp.float32)
    x_pad = jnp.concatenate(
        [x, jnp.zeros((_N_ACC - N_NODES, IN_CH), jnp.float32)])

    x128 = jnp.pad(x, ((0, 0), (0, 128 - IN_CH)))
    col2d = col_p.reshape(_E_PAD // _CHUNK, _CHUNK)
    dump = _N_HALF + (row_p & (_N_DUMP - 1))
    r_c0 = jnp.where(row_p < _N_HALF, row_p, dump)
    r_c1 = jnp.where(row_p >= _N_HALF, row_p - _N_HALF, dump)
    row2d = jnp.stack([r_c0, r_c1]).reshape(_NC * (_E_PAD // _CHUNK), _CHUNK)

    msgs = []
    for m in range(_N_CHUNKS):
        col2d_m = lax.slice_in_dim(col2d, m * (_CE // _CHUNK),
                                   (m + 1) * (_CE // _CHUNK), axis=0)
        ps_m = lax.slice_in_dim(pseudo_p, m * _CE, (m + 1) * _CE, axis=0)
        x_j_m = _sc_gather(x128, col2d_m, _CE)
        msgs.append(_tc_messages(ps_m, x_j_m, W1, b1, W2m, b2m, exp, _CE))

    parts = _sc_scatter(msgs, row2d, zeros)
    out = _tc_combine(parts.reshape(_N_ACC, 128), x_pad, root, bias)
    return out[:N_NODES]


# speedup vs baseline: 1.0059x; 1.0059x over previous
import functools

import jax
import jax.numpy as jnp
from jax import lax
from jax.experimental import pallas as pl
from jax.experimental.pallas import tpu as pltpu
from jax.experimental.pallas import tpu_sc as plsc

N_NODES = 10000
E_EDGES = 160000
IN_CH = 32
OUT_CH = 32
D_EDGE = 16
HID = 128

_NC = 2
_NS = 16
_NW = _NC * _NS
_CHUNK = 128
_E_PAD = 163840
_N_CHUNKS = 5
_CE = _E_PAD // _N_CHUNKS
_N_ACC = 10240
_N_HALF = _N_ACC // _NC
_N_DUMP = 8
_ROWS_PER_SUB = _N_HALF // _NS

_T_E = 256
_T_N = 1024


def _sc_gather(x128, col2d, n_edges):
    nch = n_edges // (_NW * _CHUNK)
    ew = nch * _CHUNK
    mesh = plsc.VectorSubcoreMesh(core_axis_name="c", subcore_axis_name="s")

    @functools.partial(
        pl.kernel,
        mesh=mesh,
        out_type=jax.ShapeDtypeStruct((n_edges, 128), jnp.float32),
        scratch_types=[
            pltpu.VMEM((nch, _CHUNK), jnp.int32),
            pltpu.VMEM((2, _CHUNK, 128), jnp.float32),
            pltpu.SemaphoreType.DMA,
            pltpu.SemaphoreType.DMA,
            pltpu.SemaphoreType.DMA,
        ],
    )
    def gather_kernel(x_hbm, col_hbm, out_hbm, idx_a, rows2, sem_g, sem_w0, sem_w1):
        wid = lax.axis_index("s") * _NC + lax.axis_index("c")
        base = wid * ew
        pltpu.sync_copy(col_hbm.at[pl.ds(wid * nch, nch)], idx_a)
        sems = (sem_w0, sem_w1)

        def step(j, b):
            @pl.when(j >= 2)
            def _():
                pltpu.make_async_copy(
                    rows2.at[b], out_hbm.at[pl.ds(base, _CHUNK)], sems[b]).wait()
            pltpu.async_copy(x_hbm.at[idx_a.at[j]], rows2.at[b], sem_g).wait()
            pltpu.async_copy(
                rows2.at[b], out_hbm.at[pl.ds(base + j * _CHUNK, _CHUNK)], sems[b])

        def body(jj, carry):
            step(2 * jj, 0)
            step(2 * jj + 1, 1)
            return carry

        lax.fori_loop(0, nch // 2, body, 0)
        pltpu.make_async_copy(rows2.at[0], out_hbm.at[pl.ds(base, _CHUNK)], sem_w0).wait()
        pltpu.make_async_copy(rows2.at[1], out_hbm.at[pl.ds(base, _CHUNK)], sem_w1).wait()

    return gather_kernel(x128, col2d)


def _sc_scatter(msgs, row2d, zeros):
    nch = _CE // (_NS * _CHUNK)
    ew = nch * _CHUNK
    rows_per_chunk = _CE // _CHUNK
    nm = _N_CHUNKS
    mesh = plsc.VectorSubcoreMesh(core_axis_name="c", subcore_axis_name="s")

    @functools.partial(
        pl.kernel,
        mesh=mesh,
        out_type=jax.ShapeDtypeStruct((_NC, _N_HALF, 128), jnp.float32),
        scratch_types=[
            pltpu.VMEM((nm * nch, _CHUNK), jnp.int32),
            pltpu.VMEM((4, _CHUNK, 128), jnp.float32),
            pltpu.VMEM_SHARED((_N_HALF + _N_DUMP, 128), jnp.float32),
        ] + [pltpu.SemaphoreType.DMA] * 8,
    )
    def scatter_kernel(m0, m1, m2, m3, m4, row_hbm, z_hbm, out_hbm,
                       idx_a, msg4, acc_sh,
                       sl0, sl1, sl2, sl3, sa0, sa1, sa2, sa3):
        msg_refs = (m0, m1, m2, m3, m4)
        sem_l = (sl0, sl1, sl2, sl3)
        sem_a = (sa0, sa1, sa2, sa3)
        cid = lax.axis_index("c")
        sid = lax.axis_index("s")
        r0 = sid * _ROWS_PER_SUB
        ebase = sid * ew
        pltpu.sync_copy(z_hbm.at[pl.ds(r0, _ROWS_PER_SUB)],
                        acc_sh.at[pl.ds(r0, _ROWS_PER_SUB)])
        rbase = cid * (nm * rows_per_chunk)
        for m in range(nm):
            pltpu.sync_copy(
                row_hbm.at[pl.ds(rbase + m * rows_per_chunk + sid * nch, nch)],
                idx_a.at[pl.ds(m * nch, nch)])
        plsc.subcore_barrier()

        def start_load(mref, t, b):
            pltpu.async_copy(mref.at[pl.ds(ebase + t * _CHUNK, _CHUNK)],
                             msg4.at[b], sem_l[b])

        def wait_load(b):
            pltpu.make_async_copy(m0.at[pl.ds(0, _CHUNK)], msg4.at[b],
                                  sem_l[b]).wait()

        def start_add(m, t2, b):
            pltpu.async_copy(msg4.at[b], acc_sh.at[idx_a.at[m * nch + t2]],
                             sem_a[b], add=True)

        def wait_add(b):
            pltpu.make_async_copy(msg4.at[b], acc_sh.at[idx_a.at[0]],
                                  sem_a[b]).wait()

        for m in range(nm):
            def body(jj, carry, m=m):
                for db in range(4):
                    t = 4 * jj + db
                    if m > 0:
                        wait_add(db)
                    else:
                        @pl.when(jj >= 1)
                        def _(db=db):
                            wait_add(db)
                    start_load(msg_refs[m], t, db)
                    if db < 2:
                        bb = db + 2
                        if m > 0:
                            @pl.when(jj == 0)
                            def _(db=db, bb=bb, m=m):
                                wait_load(bb)
                                start_add(m - 1, nch - 2 + db, bb)
                        @pl.when(jj >= 1)
                        def _(jj_=None, db=db, bb=bb, m=m):
                            wait_load(bb)
                            start_add(m, 4 * jj + db - 2, bb)
                    else:
                        bb = db - 2
                        wait_load(bb)
                        start_add(m, 4 * jj + db - 2, bb)
                return carry

            lax.fori_loop(0, nch // 4, body, 0)

        wait_load(2)
        start_add(nm - 1, nch - 2, 2)
        wait_load(3)
        start_add(nm - 1, nch - 1, 3)
        for b in range(4):
            wait_add(b)

        plsc.subcore_barrier()
        pltpu.sync_copy(acc_sh.at[pl.ds(r0, _ROWS_PER_SUB)],
                        out_hbm.at[cid, pl.ds(r0, _ROWS_PER_SUB)])

    return scatter_kernel(msgs[0], msgs[1], msgs[2], msgs[3], msgs[4], row2d, zeros)


def _msg_body(ps_ref, xj_ref, w1_ref, b1_ref, w2_ref, b2_ref, exp_ref, out_ref):
    ps = ps_ref[...]
    h = jnp.maximum(
        jnp.dot(ps, w1_ref[...], preferred_element_type=jnp.float32) + b1_ref[...],
        0.0)
    xj = xj_ref[...][:, :IN_CH]
    xr = jnp.dot(xj.astype(jnp.bfloat16), exp_ref[...],
                 preferred_element_type=jnp.float32).astype(jnp.bfloat16)
    hr = jnp.broadcast_to(h.astype(jnp.bfloat16)[:, None, :],
                          (_T_E, IN_CH, HID)).reshape(_T_E, IN_CH * HID)
    msg = jnp.dot(xr * hr, w2_ref[...], preferred_element_type=jnp.float32)
    msg = msg + jnp.dot(xj, b2_ref[...], preferred_element_type=jnp.float32)
    out_ref[:, :OUT_CH] = msg


def _tc_messages(pseudo_c, x_j, W1, b1, W2m, b2m, exp, n_edges, ps_off=0,
                 interpret=False):
    grid = n_edges // _T_E
    return pl.pallas_call(
        _msg_body,
        grid=(grid,),
        in_specs=[
            pl.BlockSpec((_T_E, D_EDGE), lambda i: (i + ps_off, 0)),
            pl.BlockSpec((_T_E, 128), lambda i: (i, 0)),
            pl.BlockSpec((D_EDGE, HID), lambda i: (0, 0)),
            pl.BlockSpec((1, HID), lambda i: (0, 0)),
            pl.BlockSpec((IN_CH * HID, OUT_CH), lambda i: (0, 0)),
            pl.BlockSpec((IN_CH, OUT_CH), lambda i: (0, 0)),
            pl.BlockSpec((IN_CH, IN_CH * HID), lambda i: (0, 0)),
        ],
        out_specs=pl.BlockSpec((_T_E, 128), lambda i: (i, 0)),
        out_shape=jax.ShapeDtypeStruct((n_edges, 128), jnp.float32),
        interpret=interpret,
    )(pseudo_c, x_j, W1, b1.reshape(1, HID), W2m, b2m, exp)


def _combine_body(p_ref, x_ref, root_ref, bias_ref, out_ref):
    acc = p_ref[...][:, :OUT_CH]
    acc = acc + jnp.dot(x_ref[...], root_ref[...], preferred_element_type=jnp.float32)
    out_ref[...] = acc + bias_ref[...]


def _tc_combine(p, x_pad, root, bias, interpret=False):
    grid = _N_ACC // _T_N
    return pl.pallas_call(
        _combine_body,
        grid=(grid,),
        in_specs=[
            pl.BlockSpec((_T_N, 128), lambda i: (i, 0)),
            pl.BlockSpec((_T_N, IN_CH), lambda i: (i, 0)),
            pl.BlockSpec((IN_CH, OUT_CH), lambda i: (0, 0)),
            pl.BlockSpec((1, OUT_CH), lambda i: (0, 0)),
        ],
        out_specs=pl.BlockSpec((_T_N, OUT_CH), lambda i: (i, 0)),
        out_shape=jax.ShapeDtypeStruct((_N_ACC, OUT_CH), jnp.float32),
        interpret=interpret,
    )(p, x_pad, root, bias.reshape(1, OUT_CH))


def kernel(x, edge_index, pseudo, W1, b1, W2, b2, root, bias):
    row = edge_index[0]
    col = edge_index[1]
    pad_e = _E_PAD - E_EDGES
    col_p = jnp.concatenate([col, jnp.zeros((pad_e,), jnp.int32)])
    row_p = jnp.concatenate([row, jnp.full((pad_e,), N_NODES, jnp.int32)])
    pseudo_tail = jnp.concatenate(
        [lax.slice_in_dim(pseudo, (_N_CHUNKS - 1) * _CE, E_EDGES, axis=0),
         jnp.zeros((pad_e, D_EDGE), jnp.float32)])
    W2m = (W2.reshape(HID, IN_CH, OUT_CH).transpose(1, 0, 2)
           .reshape(IN_CH * HID, OUT_CH).astype(jnp.bfloat16))
    b2m = b2.reshape(IN_CH, OUT_CH)
    exp = (jnp.arange(IN_CH * HID, dtype=jnp.int32)[None, :] // HID
           == jnp.arange(IN_CH, dtype=jnp.int32)[:, None]).astype(jnp.bfloat16)
    zeros = jnp.zeros((_N_HALF, 128), jnp.float32)
    x_pad = jnp.concatenate(
        [x, jnp.zeros((_N_ACC - N_NODES, IN_CH), jnp.float32)])

    x128 = jnp.pad(x, ((0, 0), (0, 128 - IN_CH)))
    col2d = col_p.reshape(_E_PAD // _CHUNK, _CHUNK)
    dump = _N_HALF + (row_p & (_N_DUMP - 1))
    r_c0 = jnp.where(row_p < _N_HALF, row_p, dump)
    r_c1 = jnp.where(row_p >= _N_HALF, row_p - _N_HALF, dump)
    row2d = jnp.stack([r_c0, r_c1]).reshape(_NC * (_E_PAD // _CHUNK), _CHUNK)

    msgs = []
    for m in range(_N_CHUNKS):
        col2d_m = lax.slice_in_dim(col2d, m * (_CE // _CHUNK),
                                   (m + 1) * (_CE // _CHUNK), axis=0)
        x_j_m = _sc_gather(x128, col2d_m, _CE)
        if m < _N_CHUNKS - 1:
            msgs.append(_tc_messages(pseudo, x_j_m, W1, b1, W2m, b2m, exp,
                                     _CE, ps_off=m * (_CE // _T_E)))
        else:
            msgs.append(_tc_messages(pseudo_tail, x_j_m, W1, b1, W2m, b2m,
                                     exp, _CE))

    parts = _sc_scatter(msgs, row2d, zeros)
    out = _tc_combine(parts.reshape(_N_ACC, 128), x_pad, root, bias)
    return out[:N_NODES]


# speedup vs baseline: 1.0229x; 1.0169x over previous
import functools

import jax
import jax.numpy as jnp
from jax import lax
from jax.experimental import pallas as pl
from jax.experimental.pallas import tpu as pltpu
from jax.experimental.pallas import tpu_sc as plsc

N_NODES = 10000
E_EDGES = 160000
IN_CH = 32
OUT_CH = 32
D_EDGE = 16
HID = 128

_NC = 2
_NS = 16
_NW = _NC * _NS
_CHUNK = 128
_E_PAD = 163840
_N_CHUNKS = 5
_CE = _E_PAD // _N_CHUNKS
_N_ACC = 10240
_N_HALF = _N_ACC // _NC
_N_DUMP = 8
_ROWS_PER_SUB = _N_HALF // _NS

_T_E = 256
_T_N = 1024


def _sc_gather(x128, col2d, n_edges):
    nch = n_edges // (_NW * _CHUNK)
    ew = nch * _CHUNK
    mesh = plsc.VectorSubcoreMesh(core_axis_name="c", subcore_axis_name="s")

    @functools.partial(
        pl.kernel,
        mesh=mesh,
        out_type=jax.ShapeDtypeStruct((n_edges, 128), jnp.float32),
        scratch_types=[
            pltpu.VMEM((nch, _CHUNK), jnp.int32),
            pltpu.VMEM((2, _CHUNK, 128), jnp.float32),
            pltpu.SemaphoreType.DMA,
            pltpu.SemaphoreType.DMA,
            pltpu.SemaphoreType.DMA,
        ],
    )
    def gather_kernel(x_hbm, col_hbm, out_hbm, idx_a, rows2, sem_g, sem_w0, sem_w1):
        wid = lax.axis_index("s") * _NC + lax.axis_index("c")
        base = wid * ew
        pltpu.sync_copy(col_hbm.at[pl.ds(wid * nch, nch)], idx_a)
        sems = (sem_w0, sem_w1)

        def step(j, b):
            @pl.when(j >= 2)
            def _():
                pltpu.make_async_copy(
                    rows2.at[b], out_hbm.at[pl.ds(base, _CHUNK)], sems[b]).wait()
            pltpu.async_copy(x_hbm.at[idx_a.at[j]], rows2.at[b], sem_g).wait()
            pltpu.async_copy(
                rows2.at[b], out_hbm.at[pl.ds(base + j * _CHUNK, _CHUNK)], sems[b])

        def body(jj, carry):
            step(2 * jj, 0)
            step(2 * jj + 1, 1)
            return carry

        lax.fori_loop(0, nch // 2, body, 0)
        pltpu.make_async_copy(rows2.at[0], out_hbm.at[pl.ds(base, _CHUNK)], sem_w0).wait()
        pltpu.make_async_copy(rows2.at[1], out_hbm.at[pl.ds(base, _CHUNK)], sem_w1).wait()

    return gather_kernel(x128, col2d)


def _sc_scatter(msgs, row2d, zeros):
    nch = _CE // (_NW * _CHUNK)
    ew = nch * _CHUNK
    rows_per_chunk = _CE // _CHUNK
    mesh = plsc.VectorSubcoreMesh(core_axis_name="c", subcore_axis_name="s")

    @functools.partial(
        pl.kernel,
        mesh=mesh,
        out_type=jax.ShapeDtypeStruct((_NC, _N_ACC, 128), jnp.float32),
        scratch_types=[
            pltpu.VMEM((nch, _CHUNK), jnp.int32),
            pltpu.VMEM((2, _CHUNK, 128), jnp.float32),
            pltpu.VMEM_SHARED((_N_ACC, 128), jnp.float32),
            pltpu.SemaphoreType.DMA,
            pltpu.SemaphoreType.DMA,
        ],
    )
    def scatter_kernel(m0, m1, m2, m3, m4, row_hbm, z_hbm, out_hbm,
                       idx_a, msg2, acc_sh, sem_l0, sem_l1):
        cid = lax.axis_index("c")
        sid = lax.axis_index("s")
        wid = sid * _NC + cid
        r0 = sid * (_N_ACC // _NS)
        rps = _N_ACC // _NS
        pltpu.sync_copy(z_hbm.at[pl.ds(r0, rps)], acc_sh.at[pl.ds(r0, rps)])
        plsc.subcore_barrier()
        sems = (sem_l0, sem_l1)
        ebase = wid * ew

        for m, msg_hbm in enumerate((m0, m1, m2, m3, m4)):
            pltpu.sync_copy(
                row_hbm.at[pl.ds(m * rows_per_chunk + wid * nch, nch)], idx_a)
            pltpu.async_copy(msg_hbm.at[pl.ds(ebase, _CHUNK)], msg2.at[0], sem_l0)
            pltpu.async_copy(msg_hbm.at[pl.ds(ebase + _CHUNK, _CHUNK)],
                             msg2.at[1], sem_l1)

            def step(j, b):
                pltpu.make_async_copy(
                    msg_hbm.at[pl.ds(ebase, _CHUNK)], msg2.at[b], sems[b]).wait()
                pltpu.sync_copy(msg2.at[b], acc_sh.at[idx_a.at[j]], add=True)

                @pl.when(j + 2 < nch)
                def _():
                    pltpu.async_copy(
                        msg_hbm.at[pl.ds(ebase + (j + 2) * _CHUNK, _CHUNK)],
                        msg2.at[b], sems[b])

            def body(jj, carry):
                step(2 * jj, 0)
                step(2 * jj + 1, 1)
                return carry

            lax.fori_loop(0, nch // 2, body, 0)

        plsc.subcore_barrier()
        pltpu.sync_copy(acc_sh.at[pl.ds(r0, rps)],
                        out_hbm.at[cid, pl.ds(r0, rps)])

    return scatter_kernel(msgs[0], msgs[1], msgs[2], msgs[3], msgs[4], row2d, zeros)


def _msg_body(ps_ref, xj_ref, w1_ref, b1_ref, w2_ref, b2_ref, exp_ref, out_ref):
    ps = ps_ref[...]
    h = jnp.maximum(
        jnp.dot(ps, w1_ref[...], preferred_element_type=jnp.float32) + b1_ref[...],
        0.0)
    xj = xj_ref[...][:, :IN_CH]
    xr = jnp.dot(xj.astype(jnp.bfloat16), exp_ref[...],
                 preferred_element_type=jnp.float32).astype(jnp.bfloat16)
    hr = jnp.broadcast_to(h.astype(jnp.bfloat16)[:, None, :],
                          (_T_E, IN_CH, HID)).reshape(_T_E, IN_CH * HID)
    msg = jnp.dot(xr * hr, w2_ref[...], preferred_element_type=jnp.float32)
    msg = msg + jnp.dot(xj, b2_ref[...], preferred_element_type=jnp.float32)
    out_ref[:, :OUT_CH] = msg


def _tc_messages(pseudo_c, x_j, W1, b1, W2m, b2m, exp, n_edges, ps_off=0,
                 interpret=False):
    grid = n_edges // _T_E
    return pl.pallas_call(
        _msg_body,
        grid=(grid,),
        in_specs=[
            pl.BlockSpec((_T_E, D_EDGE), lambda i: (i + ps_off, 0)),
            pl.BlockSpec((_T_E, 128), lambda i: (i, 0)),
            pl.BlockSpec((D_EDGE, HID), lambda i: (0, 0)),
            pl.BlockSpec((1, HID), lambda i: (0, 0)),
            pl.BlockSpec((IN_CH * HID, OUT_CH), lambda i: (0, 0)),
            pl.BlockSpec((IN_CH, OUT_CH), lambda i: (0, 0)),
            pl.BlockSpec((IN_CH, IN_CH * HID), lambda i: (0, 0)),
        ],
        out_specs=pl.BlockSpec((_T_E, 128), lambda i: (i, 0)),
        out_shape=jax.ShapeDtypeStruct((n_edges, 128), jnp.float32),
        interpret=interpret,
    )(pseudo_c, x_j, W1, b1.reshape(1, HID), W2m, b2m, exp)


def _combine_body(p0_ref, p1_ref, x_ref, root_ref, bias_ref, out_ref):
    acc = p0_ref[...][:, :OUT_CH] + p1_ref[...][:, :OUT_CH]
    acc = acc + jnp.dot(x_ref[...], root_ref[...], preferred_element_type=jnp.float32)
    out_ref[...] = acc + bias_ref[...]


def _tc_combine(p0, p1, x_pad, root, bias, interpret=False):
    grid = _N_ACC // _T_N
    return pl.pallas_call(
        _combine_body,
        grid=(grid,),
        in_specs=[
            pl.BlockSpec((_T_N, 128), lambda i: (i, 0)),
            pl.BlockSpec((_T_N, 128), lambda i: (i, 0)),
            pl.BlockSpec((_T_N, IN_CH), lambda i: (i, 0)),
            pl.BlockSpec((IN_CH, OUT_CH), lambda i: (0, 0)),
            pl.BlockSpec((1, OUT_CH), lambda i: (0, 0)),
        ],
        out_specs=pl.BlockSpec((_T_N, OUT_CH), lambda i: (i, 0)),
        out_shape=jax.ShapeDtypeStruct((_N_ACC, OUT_CH), jnp.float32),
        interpret=interpret,
    )(p0, p1, x_pad, root, bias.reshape(1, OUT_CH))


def kernel(x, edge_index, pseudo, W1, b1, W2, b2, root, bias):
    row = edge_index[0]
    col = edge_index[1]
    pad_e = _E_PAD - E_EDGES
    col_p = jnp.concatenate([col, jnp.zeros((pad_e,), jnp.int32)])
    row_p = jnp.concatenate([row, jnp.full((pad_e,), N_NODES, jnp.int32)])
    pseudo_tail = jnp.concatenate(
        [lax.slice_in_dim(pseudo, (_N_CHUNKS - 1) * _CE, E_EDGES, axis=0),
         jnp.zeros((pad_e, D_EDGE), jnp.float32)])
    W2m = (W2.reshape(HID, IN_CH, OUT_CH).transpose(1, 0, 2)
           .reshape(IN_CH * HID, OUT_CH).astype(jnp.bfloat16))
    b2m = b2.reshape(IN_CH, OUT_CH)
    exp = (jnp.arange(IN_CH * HID, dtype=jnp.int32)[None, :] // HID
           == jnp.arange(IN_CH, dtype=jnp.int32)[:, None]).astype(jnp.bfloat16)
    zeros = jnp.zeros((_N_ACC, 128), jnp.float32)
    x_pad = jnp.concatenate(
        [x, jnp.zeros((_N_ACC - N_NODES, IN_CH), jnp.float32)])

    x128 = jnp.pad(x, ((0, 0), (0, 128 - IN_CH)))
    col2d = col_p.reshape(_E_PAD // _CHUNK, _CHUNK)
    row2d = row_p.reshape(_E_PAD // _CHUNK, _CHUNK)

    msgs = []
    for m in range(_N_CHUNKS):
        col2d_m = lax.slice_in_dim(col2d, m * (_CE // _CHUNK),
                                   (m + 1) * (_CE // _CHUNK), axis=0)
        x_j_m = _sc_gather(x128, col2d_m, _CE)
        if m < _N_CHUNKS - 1:
            msgs.append(_tc_messages(pseudo, x_j_m, W1, b1, W2m, b2m, exp,
                                     _CE, ps_off=m * (_CE // _T_E)))
        else:
            msgs.append(_tc_messages(pseudo_tail, x_j_m, W1, b1, W2m, b2m,
                                     exp, _CE))

    parts = _sc_scatter(msgs, row2d, zeros)
    out = _tc_combine(parts[0], parts[1], x_pad, root, bias)
    return out[:N_NODES]


# speedup vs baseline: 1.4843x; 1.4510x over previous
import functools

import jax
import jax.numpy as jnp
from jax import lax
from jax.experimental import pallas as pl
from jax.experimental.pallas import tpu as pltpu
from jax.experimental.pallas import tpu_sc as plsc

N_NODES = 10000
E_EDGES = 160000
IN_CH = 32
OUT_CH = 32
D_EDGE = 16
HID = 128

_NC = 2
_NS = 16
_NW = _NC * _NS
_CHUNK = 128
_E_PAD = 163840
_N_CHUNKS = 5
_CE = _E_PAD // _N_CHUNKS
_N_ACC = 10240
_N_HALF = _N_ACC // _NC
_N_DUMP = 8
_ROWS_PER_SUB = _N_HALF // _NS

_T_E = 256
_T_N = 1024


def _sc_gather(x128, col2d, n_edges):
    nch = n_edges // (_NW * _CHUNK)
    ew = nch * _CHUNK
    mesh = plsc.VectorSubcoreMesh(core_axis_name="c", subcore_axis_name="s")

    @functools.partial(
        pl.kernel,
        mesh=mesh,
        out_type=jax.ShapeDtypeStruct((n_edges, 128), jnp.float32),
        scratch_types=[
            pltpu.VMEM((nch, _CHUNK), jnp.int32),
            pltpu.VMEM((2, _CHUNK, 128), jnp.float32),
            pltpu.SemaphoreType.DMA,
            pltpu.SemaphoreType.DMA,
            pltpu.SemaphoreType.DMA,
        ],
    )
    def gather_kernel(x_hbm, col_hbm, out_hbm, idx_a, rows2, sem_g, sem_w0, sem_w1):
        wid = lax.axis_index("s") * _NC + lax.axis_index("c")
        base = wid * ew
        pltpu.sync_copy(col_hbm.at[pl.ds(wid * nch, nch)], idx_a)
        sems = (sem_w0, sem_w1)

        def step(j, b):
            @pl.when(j >= 2)
            def _():
                pltpu.make_async_copy(
                    rows2.at[b], out_hbm.at[pl.ds(base, _CHUNK)], sems[b]).wait()
            pltpu.async_copy(x_hbm.at[idx_a.at[j]], rows2.at[b], sem_g).wait()
            pltpu.async_copy(
                rows2.at[b], out_hbm.at[pl.ds(base + j * _CHUNK, _CHUNK)], sems[b])

        def body(jj, carry):
            step(2 * jj, 0)
            step(2 * jj + 1, 1)
            return carry

        lax.fori_loop(0, nch // 2, body, 0)
        pltpu.make_async_copy(rows2.at[0], out_hbm.at[pl.ds(base, _CHUNK)], sem_w0).wait()
        pltpu.make_async_copy(rows2.at[1], out_hbm.at[pl.ds(base, _CHUNK)], sem_w1).wait()

    return gather_kernel(x128, col2d)


def _sc_scatter(msgs, row2d, zeros):
    nch = _CE // (_NW * _CHUNK)
    ew = nch * _CHUNK
    rows_per_chunk = _CE // _CHUNK
    mesh = plsc.VectorSubcoreMesh(core_axis_name="c", subcore_axis_name="s")

    @functools.partial(
        pl.kernel,
        mesh=mesh,
        out_type=jax.ShapeDtypeStruct((_NC, _N_ACC, 128), jnp.float32),
        scratch_types=[
            pltpu.VMEM((nch, _CHUNK), jnp.int32),
            pltpu.VMEM((2, _CHUNK, 128), jnp.float32),
            pltpu.VMEM_SHARED((_N_ACC, 128), jnp.float32),
            pltpu.SemaphoreType.DMA,
            pltpu.SemaphoreType.DMA,
        ],
    )
    def scatter_kernel(m0, m1, m2, m3, m4, row_hbm, z_hbm, out_hbm,
                       idx_a, msg2, acc_sh, sem_l0, sem_l1):
        cid = lax.axis_index("c")
        sid = lax.axis_index("s")
        wid = sid * _NC + cid
        r0 = sid * (_N_ACC // _NS)
        rps = _N_ACC // _NS
        pltpu.sync_copy(z_hbm.at[pl.ds(r0, rps)], acc_sh.at[pl.ds(r0, rps)])
        plsc.subcore_barrier()
        sems = (sem_l0, sem_l1)
        ebase = wid * ew

        for m, msg_hbm in enumerate((m0, m1, m2, m3, m4)):
            pltpu.sync_copy(
                row_hbm.at[pl.ds(m * rows_per_chunk + wid * nch, nch)], idx_a)
            pltpu.async_copy(msg_hbm.at[pl.ds(ebase, _CHUNK)], msg2.at[0], sem_l0)
            pltpu.async_copy(msg_hbm.at[pl.ds(ebase + _CHUNK, _CHUNK)],
                             msg2.at[1], sem_l1)

            def step(j, b):
                pltpu.make_async_copy(
                    msg_hbm.at[pl.ds(ebase, _CHUNK)], msg2.at[b], sems[b]).wait()
                pltpu.sync_copy(msg2.at[b], acc_sh.at[idx_a.at[j]], add=True)

                @pl.when(j + 2 < nch)
                def _():
                    pltpu.async_copy(
                        msg_hbm.at[pl.ds(ebase + (j + 2) * _CHUNK, _CHUNK)],
                        msg2.at[b], sems[b])

            def body(jj, carry):
                step(2 * jj, 0)
                step(2 * jj + 1, 1)
                return carry

            lax.fori_loop(0, nch // 2, body, 0)

        plsc.subcore_barrier()
        pltpu.sync_copy(acc_sh.at[pl.ds(r0, rps)],
                        out_hbm.at[cid, pl.ds(r0, rps)])

    return scatter_kernel(msgs[0], msgs[1], msgs[2], msgs[3], msgs[4], row2d, zeros)


def _msg_body(ps_ref, xj_ref, w1_ref, b1_ref, w2_ref, b2_ref, exp_ref, out_ref):
    ps = ps_ref[...]
    h = jnp.maximum(
        jnp.dot(ps, w1_ref[...], preferred_element_type=jnp.float32) + b1_ref[...],
        0.0)
    xj = xj_ref[...][:, :IN_CH]
    g = jnp.dot(h.astype(jnp.bfloat16), w2_ref[...],
                preferred_element_type=jnp.float32)
    xje = jnp.dot(xj.astype(jnp.bfloat16), exp_ref[...],
                  preferred_element_type=jnp.float32)
    p = xje * g
    p = p[:, :512] + p[:, 512:]
    p = p[:, :256] + p[:, 256:]
    p = p[:, :128] + p[:, 128:]
    p = p[:, :64] + p[:, 64:]
    msg = p[:, :OUT_CH] + p[:, OUT_CH:]
    msg = msg + jnp.dot(xj, b2_ref[...], preferred_element_type=jnp.float32)
    out_ref[:, :OUT_CH] = msg


def _tc_messages(pseudo_c, x_j, W1, b1, W2b, b2m, exp, n_edges, ps_off=0,
                 interpret=False):
    grid = n_edges // _T_E
    return pl.pallas_call(
        _msg_body,
        grid=(grid,),
        in_specs=[
            pl.BlockSpec((_T_E, D_EDGE), lambda i: (i + ps_off, 0)),
            pl.BlockSpec((_T_E, 128), lambda i: (i, 0)),
            pl.BlockSpec((D_EDGE, HID), lambda i: (0, 0)),
            pl.BlockSpec((1, HID), lambda i: (0, 0)),
            pl.BlockSpec((HID, IN_CH * OUT_CH), lambda i: (0, 0)),
            pl.BlockSpec((IN_CH, OUT_CH), lambda i: (0, 0)),
            pl.BlockSpec((IN_CH, IN_CH * OUT_CH), lambda i: (0, 0)),
        ],
        out_specs=pl.BlockSpec((_T_E, 128), lambda i: (i, 0)),
        out_shape=jax.ShapeDtypeStruct((n_edges, 128), jnp.float32),
        interpret=interpret,
    )(pseudo_c, x_j, W1, b1.reshape(1, HID), W2b, b2m, exp)


def _combine_body(p0_ref, p1_ref, x_ref, root_ref, bias_ref, out_ref):
    acc = p0_ref[...][:, :OUT_CH] + p1_ref[...][:, :OUT_CH]
    acc = acc + jnp.dot(x_ref[...], root_ref[...], preferred_element_type=jnp.float32)
    out_ref[...] = acc + bias_ref[...]


def _tc_combine(p0, p1, x_pad, root, bias, interpret=False):
    grid = _N_ACC // _T_N
    return pl.pallas_call(
        _combine_body,
        grid=(grid,),
        in_specs=[
            pl.BlockSpec((_T_N, 128), lambda i: (i, 0)),
            pl.BlockSpec((_T_N, 128), lambda i: (i, 0)),
            pl.BlockSpec((_T_N, IN_CH), lambda i: (i, 0)),
            pl.BlockSpec((IN_CH, OUT_CH), lambda i: (0, 0)),
            pl.BlockSpec((1, OUT_CH), lambda i: (0, 0)),
        ],
        out_specs=pl.BlockSpec((_T_N, OUT_CH), lambda i: (i, 0)),
        out_shape=jax.ShapeDtypeStruct((_N_ACC, OUT_CH), jnp.float32),
        interpret=interpret,
    )(p0, p1, x_pad, root, bias.reshape(1, OUT_CH))


def kernel(x, edge_index, pseudo, W1, b1, W2, b2, root, bias):
    row = edge_index[0]
    col = edge_index[1]
    pad_e = _E_PAD - E_EDGES
    col_p = jnp.concatenate([col, jnp.zeros((pad_e,), jnp.int32)])
    row_p = jnp.concatenate([row, jnp.full((pad_e,), N_NODES, jnp.int32)])
    pseudo_tail = jnp.concatenate(
        [lax.slice_in_dim(pseudo, (_N_CHUNKS - 1) * _CE, E_EDGES, axis=0),
         jnp.zeros((pad_e, D_EDGE), jnp.float32)])
    W2b = W2.astype(jnp.bfloat16)
    b2m = b2.reshape(IN_CH, OUT_CH)
    exp = (jnp.arange(IN_CH * OUT_CH, dtype=jnp.int32)[None, :] // OUT_CH
           == jnp.arange(IN_CH, dtype=jnp.int32)[:, None]).astype(jnp.bfloat16)
    zeros = jnp.zeros((_N_ACC, 128), jnp.float32)
    x_pad = jnp.concatenate(
        [x, jnp.zeros((_N_ACC - N_NODES, IN_CH), jnp.float32)])

    x128 = jnp.pad(x, ((0, 0), (0, 128 - IN_CH)))
    col2d = col_p.reshape(_E_PAD // _CHUNK, _CHUNK)
    row2d = row_p.reshape(_E_PAD // _CHUNK, _CHUNK)

    msgs = []
    for m in range(_N_CHUNKS):
        col2d_m = lax.slice_in_dim(col2d, m * (_CE // _CHUNK),
                                   (m + 1) * (_CE // _CHUNK), axis=0)
        x_j_m = _sc_gather(x128, col2d_m, _CE)
        if m < _N_CHUNKS - 1:
            msgs.append(_tc_messages(pseudo, x_j_m, W1, b1, W2b, b2m, exp,
                                     _CE, ps_off=m * (_CE // _T_E)))
        else:
            msgs.append(_tc_messages(pseudo_tail, x_j_m, W1, b1, W2b, b2m,
                                     exp, _CE))

    parts = _sc_scatter(msgs, row2d, zeros)
    out = _tc_combine(parts[0], parts[1], x_pad, root, bias)
    return out[:N_NODES]
